# baseline (speedup 1.0000x reference)
import jax
import jax.numpy as jnp
import numpy as _np
from jax import lax
from jax.experimental import pallas as pl
from jax.experimental.pallas import tpu as pltpu
from jax.experimental.pallas import tpu_sc as plsc

N = 10000
E = 320000
H = 128
NH = 8
DK = 16
NQ = 5000
MH = 64

NC = 2
NS = 16
L = 16
HPC = NH // NC

CH = 48
EP = 344064
EPW = EP // NS
NCHUNK = EPW // CH
NITER = NCHUNK // 2
NGROUP = CH // L
NP = 10240
NPW = NP // 2
NZR = NP // 32
NA = 5504
RPS = NA // NS


def _qkv_body(x_ref, wq_ref, bq_ref, wk_ref, wv_ref, q_ref, kv_ref):
    x = x_ref[...]
    dn = (((1,), (1,)), ((), ()))
    q_ref[...] = lax.dot_general(
        x, wq_ref[...], dn, preferred_element_type=jnp.float32) + bq_ref[...]
    for c in (0, 1):
        sl = pl.ds(c * MH, MH)
        rows = pl.ds(c * N, N)
        kv_ref[rows, 0:MH] = lax.dot_general(
            x, wk_ref[sl, :], dn, preferred_element_type=jnp.float32)
        kv_ref[rows, MH:H] = lax.dot_general(
            x, wv_ref[sl, :], dn, preferred_element_type=jnp.float32)


def _qkv_call(x, Wq, bq, Wk, Wv):
    out = [jax.ShapeDtypeStruct((N, H), jnp.float32),
           jax.ShapeDtypeStruct((2 * N, H), jnp.float32)]
    return pl.pallas_call(_qkv_body, out_shape=out)(x, Wq, bq.reshape(1, H), Wk, Wv)


def _edge_body(src_hbm, dst_hbm, kv_hbm, q_hbm, zero_hbm, out_hbm,
               src_all, dst_all,
               ia, oa, kvr_a, qr_a, kvr_b, qr_b,
               sk_a, sq_a, sk_b, sq_b, ss, acc):
    cid = lax.axis_index("c")
    sid = lax.axis_index("s")

    pltpu.sync_copy(zero_hbm.at[pl.ds(sid * RPS, RPS)],
                    acc.at[pl.ds(sid * RPS, RPS)])

    wbase = sid * EPW
    pltpu.sync_copy(src_hbm.at[pl.ds(wbase, EPW)], src_all)
    pltpu.sync_copy(dst_hbm.at[pl.ds(wbase, EPW)], dst_all)
    roff = cid * N

    def _adj(j, carry):
        sl = pl.ds(j * L, L)
        src_all[sl] = src_all[sl] + roff
        return carry

    lax.fori_loop(0, EPW // L, _adj, 0)
    qcol0 = cid * MH

    def _zrow(i, carry):
        for j in range(H // L):
            oa[i, pl.ds(j * L, L)] = jnp.zeros((L,), jnp.float32)
        return carry

    lax.fori_loop(0, 2 * CH, _zrow, 0)

    bufs = ((kvr_a, qr_a, sk_a, sq_a),
            (kvr_b, qr_b, sk_b, sq_b))

    def _issue_gathers(c, b):
        kvr, qr, sk, sq = bufs[b]
        off = pl.multiple_of(c * CH, 16)
        pltpu.async_copy(kv_hbm.at[src_all.at[pl.ds(off, CH)]], kvr, sk)
        pltpu.async_copy(q_hbm.at[dst_all.at[pl.ds(off, CH)]], qr, sq)

    _issue_gathers(0, 0)
    _issue_gathers(1, 1)
    plsc.subcore_barrier()

    def _phase(i, b, c):
        kvr, qr, sk, sq = bufs[b]
        coff = pl.multiple_of(c * CH, 16)

        @pl.when(c > 0)
        def _():
            pltpu.make_async_copy(oa, acc.at[ia], ss).wait()
            poff = pl.multiple_of((c - 1) * CH, 16)

            def _zg(g, gcarry):
                zlanes = CH + g * L + lax.iota(jnp.int32, L)
                dstv = dst_all[pl.ds(poff + g * L, L)]
                colz0 = lax.shift_left(dstv & 31, 2)
                zv = jnp.zeros((L,), jnp.float32)
                for hh in range(HPC):
                    plsc.store_scatter(oa, [zlanes, colz0 + hh], zv)
                return gcarry

            lax.fori_loop(0, NGROUP, _zg, 0)

        pltpu.make_async_copy(kv_hbm.at[src_all.at[pl.ds(coff, CH)]], kvr, sk).wait()
        pltpu.make_async_copy(q_hbm.at[dst_all.at[pl.ds(coff, CH)]], qr, sq).wait()

        def _group(g, gcarry):
            iot = lax.iota(jnp.int32, L)
            rots = [(iot + d) & (DK - 1) for d in range(DK)]
            lanes = g * L + iot
            dstv = dst_all[pl.ds(coff + g * L, L)]
            ia[pl.ds(g * L, L)] = lax.shift_right_logical(dstv, 1)
            ia[pl.ds(CH + g * L, L)] = NPW + lax.shift_right_logical(dstv, 5)
            colw0 = lax.shift_left(dstv & 1, 6)
            colz0 = lax.shift_left(dstv & 31, 2)
            zlanes = lanes + CH
            zv = jnp.zeros((L,), jnp.float32)
            for hh in range(HPC):
                parts = []
                for j in range(4):
                    pacc = jnp.zeros((L,), jnp.float32)
                    for dd in range(4):
                        d = j * 4 + dd
                        colv = rots[d] + (hh * DK)
                        kv = plsc.load_gather(kvr, [lanes, colv])
                        qv = plsc.load_gather(qr, [lanes, colv + qcol0])
                        pacc = pacc + kv * qv
                    parts.append(pacc)
                accv = (parts[0] + parts[1]) + (parts[2] + parts[3])
                es = jnp.exp(jnp.clip(accv * 0.25, -10.0, 10.0))
                plsc.store_scatter(oa, [zlanes, colz0 + hh], es)
                for d in range(DK):
                    rotc = rots[d] + (hh * DK)
                    cv16 = rotc + MH
                    colv = colw0 + rotc
                    vv = plsc.load_gather(kvr, [lanes, cv16])
                    plsc.store_scatter(oa, [lanes, colv], vv * es)
                    plsc.store_scatter(oa, [lanes, colv ^ 64], zv)
            return gcarry

        lax.fori_loop(0, NGROUP, _group, 0)

        @pl.when(i < NITER - 1)
        def _():
            _issue_gathers(c + 2, b)

        pltpu.async_copy(oa, acc.at[ia], ss, add=True)

    def _iter(i, carry):
        _phase(i, 0, 2 * i)
        _phase(i, 1, 2 * i + 1)
        return carry

    lax.fori_loop(0, NITER, _iter, 0)

    pltpu.make_async_copy(oa, acc.at[ia], ss).wait()
    plsc.subcore_barrier()

    pltpu.sync_copy(acc.at[pl.ds(sid * RPS, RPS)],
                    out_hbm.at[cid, pl.ds(sid * RPS, RPS)])


def _edge_call(src, dst, kv, q):
    zeros = jnp.zeros((NA, H), jnp.float32)
    mesh = plsc.VectorSubcoreMesh(core_axis_name="c", subcore_axis_name="s")
    dbuf = [
        pltpu.VMEM((CH, H), jnp.float32),
        pltpu.VMEM((CH, H), jnp.float32),
    ]
    fn = pl.kernel(
        _edge_body,
        out_type=jax.ShapeDtypeStruct((NC, NA, H), jnp.float32),
        mesh=mesh,
        compiler_params=pltpu.CompilerParams(needs_layout_passes=False),
        scratch_types=[
            pltpu.VMEM((EPW,), jnp.int32),
            pltpu.VMEM((EPW,), jnp.int32),
            pltpu.VMEM((2 * CH,), jnp.int32),
            pltpu.VMEM((2 * CH, H), jnp.float32),
            *dbuf, *dbuf,
            pltpu.SemaphoreType.DMA,
            pltpu.SemaphoreType.DMA,
            pltpu.SemaphoreType.DMA,
            pltpu.SemaphoreType.DMA,
            pltpu.SemaphoreType.DMA,
            pltpu.VMEM_SHARED((NA, H), jnp.float32),
        ],
    )
    return fn(src, dst, kv, q, zeros)


def _ctx_body(wv_ref, z_ref, wo_ref, bo_ref, ctx_ref):
    zinv = 1.0 / (z_ref[...] + 1e-9)
    hsel = (lax.broadcasted_iota(jnp.int32, (NH, H), 1) // DK
            == lax.broadcasted_iota(jnp.int32, (NH, H), 0)).astype(jnp.float32)
    zrep = lax.dot_general(zinv, hsel, (((1,), (0,)), ((), ())),
                           preferred_element_type=jnp.float32)
    ctx_ref[...] = lax.dot_general(
        wv_ref[...] * zrep, wo_ref[...], (((1,), (1,)), ((), ())),
        preferred_element_type=jnp.float32) + bo_ref[...]


def _loss_body(x_ref, ctx_ref, qp0_ref, qp1_ref, qp2_ref, sp_ref,
               qwn_ref, qbn_ref, qwc_ref, qbc_ref, qb_ref,
               qwan_ref, qwac_ref, qba_ref, swn_ref, sbn_ref, swc_ref,
               sbc_ref, sb_ref, swan_ref, swac_ref, sba_ref, out_ref):
    dn = (((1,), (1,)), ((), ()))

    def mm(a, b):
        return lax.dot_general(a, b, dn, preferred_element_type=jnp.float32)

    x = x_ref[...]
    ctx = ctx_ref[...]

    nh = jnp.tanh(mm(x[:NQ], qwn_ref[...]) + qbn_ref[...])
    ch = jnp.tanh(mm(ctx[:NQ], qwc_ref[...]) + qbc_ref[...])
    s = []
    for o in range(3):
        t = mm(ch, qb_ref[o])
        bil = jnp.sum(nh * t, axis=1, keepdims=True)
        aff = (mm(nh, qwan_ref[o:o + 1, :]) + mm(ch, qwac_ref[o:o + 1, :])
               + qba_ref[o:o + 1, 0:1])
        s.append(bil + aff)
    m = jnp.maximum(jnp.maximum(s[0], s[1]), s[2])
    lse = m + jnp.log(jnp.exp(s[0] - m) + jnp.exp(s[1] - m) + jnp.exp(s[2] - m))
    vr_loss = (jnp.sum(qp0_ref[...] * (lse - s[0]))
               + jnp.sum(qp1_ref[...] * (lse - s[1]))
               + jnp.sum(qp2_ref[...] * (lse - s[2])))

    nhs = jnp.tanh(mm(x[NQ:], swn_ref[...]) + sbn_ref[...])
    chs = jnp.tanh(mm(ctx[NQ:], swc_ref[...]) + sbc_ref[...])
    ts = mm(chs, sb_ref[...])
    g = (jnp.sum(nhs * ts, axis=1, keepdims=True) + mm(nhs, swan_ref[...])
         + mm(chs, swac_ref[...]) + sba_ref[0:1, 0:1])
    gp = (jnp.maximum(g, 0.0) - g * sp_ref[...]
          + jnp.log1p(jnp.exp(-jnp.abs(g))))
    out_ref[...] = jnp.reshape(vr_loss + jnp.sum(gp), (1, 1))


def _tail_call(x, wv, z, question_prob, schema_prob, Wo, bo, qWn, qbn, qWc,
               qbc, qB, qWa, qba, sWn, sbn, sWc, sbc, sB, sWa, sba):
    ctx = pl.pallas_call(
        _ctx_body, out_shape=jax.ShapeDtypeStruct((N, H), jnp.float32))(
            wv, z, Wo, bo.reshape(1, H))
    out = pl.pallas_call(
        _loss_body, out_shape=jax.ShapeDtypeStruct((1, 1), jnp.float32))(
            x, ctx,
            question_prob[:, 0:1], question_prob[:, 1:2], question_prob[:, 2:3],
            schema_prob.reshape(NQ, 1),
            qWn, qbn.reshape(1, MH), qWc, qbc.reshape(1, MH), qB,
            qWa[:, :MH], qWa[:, MH:], qba.reshape(3, 1),
            sWn, sbn.reshape(1, MH), sWc, sbc.reshape(1, MH), sB[0],
            sWa[0:1, :MH], sWa[0:1, MH:], sba.reshape(1, 1))
    return out[0, 0]


def kernel(inputs, edge_index, question_prob, schema_prob, Wq, bq, Wk, Wv, Wo,
           bo, qWn, qbn, qWc, qbc, qB, qWa, qba, sWn, sbn, sWc, sbc, sB, sWa,
           sba):
    q, kv = _qkv_call(inputs, Wq, bq, Wk, Wv)
    npad = EP - E
    src_p = jnp.concatenate([edge_index[0], jnp.zeros((npad,), jnp.int32)])
    dst_p = jnp.concatenate([edge_index[1], jnp.full((npad,), N, jnp.int32)])
    out = _edge_call(src_p, dst_p, kv, q)
    wv = jnp.concatenate(
        [out[0, :NPW].reshape(NP, MH)[:N], out[1, :NPW].reshape(NP, MH)[:N]],
        axis=1)
    z = jnp.concatenate(
        [out[0, NPW:NPW + NZR].reshape(NP, HPC)[:N],
         out[1, NPW:NPW + NZR].reshape(NP, HPC)[:N]], axis=1)
    return _tail_call(inputs, wv, z, question_prob, schema_prob,
                      Wo, bo, qWn, qbn, qWc, qbc, qB, qWa, qba, sWn, sbn,
                      sWc, sbc, sB, sWa, sba)


# speedup vs baseline: 1.5936x; 1.5936x over previous
import jax
import jax.numpy as jnp
import numpy as _np
from jax import lax
from jax.experimental import pallas as pl
from jax.experimental.pallas import tpu as pltpu
from jax.experimental.pallas import tpu_sc as plsc

N = 10000
E = 320000
H = 128
NH = 8
DK = 16
NQ = 5000
MH = 64

NC = 2
NS = 16
L = 16
HPC = NH // NC

CH = 32
EP = 327680
EPW = EP // NS
NCHUNK = EPW // CH
NITER = NCHUNK // 2
NGROUP = CH // L
NP = 10240
NPW = NP // 2
NZR = NP // 32
NA = 5504
RPS = NA // NS


def _qkv_body(x_ref, wq_ref, bq_ref, wk_ref, wv_ref, q_ref, kv_ref):
    x = x_ref[...]
    dn = (((1,), (1,)), ((), ()))
    q_ref[...] = lax.dot_general(
        x, wq_ref[...], dn, preferred_element_type=jnp.float32) + bq_ref[...]
    for c in (0, 1):
        sl = pl.ds(c * MH, MH)
        rows = pl.ds(c * N, N)
        kv_ref[rows, 0:MH] = lax.dot_general(
            x, wk_ref[sl, :], dn, preferred_element_type=jnp.float32)
        kv_ref[rows, MH:H] = lax.dot_general(
            x, wv_ref[sl, :], dn, preferred_element_type=jnp.float32)


def _qkv_call(x, Wq, bq, Wk, Wv):
    out = [jax.ShapeDtypeStruct((N, H), jnp.float32),
           jax.ShapeDtypeStruct((2 * N, H), jnp.float32)]
    return pl.pallas_call(_qkv_body, out_shape=out)(x, Wq, bq.reshape(1, H), Wk, Wv)


def _edge_body(src_hbm, dst_hbm, kv_hbm, q_hbm, zero_hbm, out_hbm,
               src_all, dst_all,
               ia_a, kvr_a, qr_a, oa_a,
               ia_b, kvr_b, qr_b, oa_b,
               sk_a, sq_a, ss_a, sk_b, sq_b, ss_b, acc):
    cid = lax.axis_index("c")
    sid = lax.axis_index("s")

    pltpu.sync_copy(zero_hbm.at[pl.ds(sid * RPS, RPS)],
                    acc.at[pl.ds(sid * RPS, RPS)])

    wbase = sid * EPW
    pltpu.sync_copy(src_hbm.at[pl.ds(wbase, EPW)], src_all)
    pltpu.sync_copy(dst_hbm.at[pl.ds(wbase, EPW)], dst_all)
    roff = cid * N

    def _adj(j, carry):
        sl = pl.ds(j * L, L)
        src_all[sl] = src_all[sl] + roff
        return carry

    lax.fori_loop(0, EPW // L, _adj, 0)
    qcol0 = cid * MH

    for _oa in (oa_a, oa_b):
        def _zrow(i, carry, _o=_oa):
            for j in range(H // L):
                _o[i, pl.ds(j * L, L)] = jnp.zeros((L,), jnp.float32)
            return carry

        lax.fori_loop(0, 2 * CH, _zrow, 0)

    bufs = ((ia_a, kvr_a, qr_a, oa_a, sk_a, sq_a, ss_a),
            (ia_b, kvr_b, qr_b, oa_b, sk_b, sq_b, ss_b))

    def _issue_gathers(c, b):
        ia, kvr, qr, oa, sk, sq, ss = bufs[b]
        off = pl.multiple_of(c * CH, CH)
        pltpu.async_copy(kv_hbm.at[src_all.at[pl.ds(off, CH)]], kvr, sk)
        pltpu.async_copy(q_hbm.at[dst_all.at[pl.ds(off, CH)]], qr, sq)

    _issue_gathers(0, 0)
    _issue_gathers(1, 1)
    plsc.subcore_barrier()

    def _phase(i, b, c):
        ia, kvr, qr, oa, sk, sq, ss = bufs[b]
        coff = pl.multiple_of(c * CH, CH)

        @pl.when(i > 0)
        def _():
            pltpu.make_async_copy(oa, acc.at[ia], ss).wait()
            poff = pl.multiple_of((c - 2) * CH, CH)

            def _zg(g, gcarry):
                zlanes = CH + g * L + lax.iota(jnp.int32, L)
                dstv = dst_all[pl.ds(poff + g * L, L)]
                colz0 = lax.shift_left(dstv & 31, 2)
                zv = jnp.zeros((L,), jnp.float32)
                for hh in range(HPC):
                    plsc.store_scatter(oa, [zlanes, colz0 + hh], zv)
                return gcarry

            lax.fori_loop(0, NGROUP, _zg, 0)

        pltpu.make_async_copy(kv_hbm.at[src_all.at[pl.ds(coff, CH)]], kvr, sk).wait()
        pltpu.make_async_copy(q_hbm.at[dst_all.at[pl.ds(coff, CH)]], qr, sq).wait()

        def _group(g, gcarry):
            iot = lax.iota(jnp.int32, L)
            rots = [(iot + d) & (DK - 1) for d in range(DK)]
            lanes = g * L + iot
            dstv = dst_all[pl.ds(coff + g * L, L)]
            ia[pl.ds(g * L, L)] = lax.shift_right_logical(dstv, 1)
            ia[pl.ds(CH + g * L, L)] = NPW + lax.shift_right_logical(dstv, 5)
            colw0 = lax.shift_left(dstv & 1, 6)
            colz0 = lax.shift_left(dstv & 31, 2)
            zlanes = lanes + CH
            zv = jnp.zeros((L,), jnp.float32)
            for hh in range(HPC):
                parts = []
                for j in range(4):
                    pacc = jnp.zeros((L,), jnp.float32)
                    for dd in range(4):
                        d = j * 4 + dd
                        colv = rots[d] + (hh * DK)
                        kv = plsc.load_gather(kvr, [lanes, colv])
                        qv = plsc.load_gather(qr, [lanes, colv + qcol0])
                        pacc = pacc + kv * qv
                    parts.append(pacc)
                accv = (parts[0] + parts[1]) + (parts[2] + parts[3])
                es = jnp.exp(jnp.clip(accv * 0.25, -10.0, 10.0))
                plsc.store_scatter(oa, [zlanes, colz0 + hh], es)
                for d in range(DK):
                    rotc = rots[d] + (hh * DK)
                    cv16 = rotc + MH
                    colv = colw0 + rotc
                    vv = plsc.load_gather(kvr, [lanes, cv16])
                    plsc.store_scatter(oa, [lanes, colv], vv * es)
                    plsc.store_scatter(oa, [lanes, colv ^ 64], zv)
            return gcarry

        lax.fori_loop(0, NGROUP, _group, 0)

        @pl.when(i < NITER - 1)
        def _():
            _issue_gathers(c + 2, b)

        pltpu.async_copy(oa, acc.at[ia], ss, add=True)

    def _iter(i, carry):
        _phase(i, 0, 2 * i)
        _phase(i, 1, 2 * i + 1)
        return carry

    lax.fori_loop(0, NITER, _iter, 0)

    pltpu.make_async_copy(oa_a, acc.at[ia_a], ss_a).wait()
    pltpu.make_async_copy(oa_b, acc.at[ia_b], ss_b).wait()
    plsc.subcore_barrier()

    pltpu.sync_copy(acc.at[pl.ds(sid * RPS, RPS)],
                    out_hbm.at[cid, pl.ds(sid * RPS, RPS)])


def _edge_call(src, dst, kv, q):
    zeros = jnp.zeros((NA, H), jnp.float32)
    mesh = plsc.VectorSubcoreMesh(core_axis_name="c", subcore_axis_name="s")
    dbuf = [
        pltpu.VMEM((2 * CH,), jnp.int32),
        pltpu.VMEM((CH, H), jnp.float32),
        pltpu.VMEM((CH, H), jnp.float32),
        pltpu.VMEM((2 * CH, H), jnp.float32),
    ]
    fn = pl.kernel(
        _edge_body,
        out_type=jax.ShapeDtypeStruct((NC, NA, H), jnp.float32),
        mesh=mesh,
        compiler_params=pltpu.CompilerParams(needs_layout_passes=False),
        scratch_types=[
            pltpu.VMEM((EPW,), jnp.int32),
            pltpu.VMEM((EPW,), jnp.int32),
            *dbuf, *dbuf,
            pltpu.SemaphoreType.DMA,
            pltpu.SemaphoreType.DMA,
            pltpu.SemaphoreType.DMA,
            pltpu.SemaphoreType.DMA,
            pltpu.SemaphoreType.DMA,
            pltpu.SemaphoreType.DMA,
            pltpu.VMEM_SHARED((NA, H), jnp.float32),
        ],
    )
    return fn(src, dst, kv, q, zeros)


def _ctx_body(wv_ref, z_ref, wo_ref, bo_ref, ctx_ref):
    zinv = 1.0 / (z_ref[...] + 1e-9)
    hsel = (lax.broadcasted_iota(jnp.int32, (NH, H), 1) // DK
            == lax.broadcasted_iota(jnp.int32, (NH, H), 0)).astype(jnp.float32)
    zrep = lax.dot_general(zinv, hsel, (((1,), (0,)), ((), ())),
                           preferred_element_type=jnp.float32)
    ctx_ref[...] = lax.dot_general(
        wv_ref[...] * zrep, wo_ref[...], (((1,), (1,)), ((), ())),
        preferred_element_type=jnp.float32) + bo_ref[...]


def _loss_body(x_ref, ctx_ref, qp0_ref, qp1_ref, qp2_ref, sp_ref,
               qwn_ref, qbn_ref, qwc_ref, qbc_ref, qb_ref,
               qwan_ref, qwac_ref, qba_ref, swn_ref, sbn_ref, swc_ref,
               sbc_ref, sb_ref, swan_ref, swac_ref, sba_ref, out_ref):
    dn = (((1,), (1,)), ((), ()))

    def mm(a, b):
        return lax.dot_general(a, b, dn, preferred_element_type=jnp.float32)

    x = x_ref[...]
    ctx = ctx_ref[...]

    nh = jnp.tanh(mm(x[:NQ], qwn_ref[...]) + qbn_ref[...])
    ch = jnp.tanh(mm(ctx[:NQ], qwc_ref[...]) + qbc_ref[...])
    s = []
    for o in range(3):
        t = mm(ch, qb_ref[o])
        bil = jnp.sum(nh * t, axis=1, keepdims=True)
        aff = (mm(nh, qwan_ref[o:o + 1, :]) + mm(ch, qwac_ref[o:o + 1, :])
               + qba_ref[o:o + 1, 0:1])
        s.append(bil + aff)
    m = jnp.maximum(jnp.maximum(s[0], s[1]), s[2])
    lse = m + jnp.log(jnp.exp(s[0] - m) + jnp.exp(s[1] - m) + jnp.exp(s[2] - m))
    vr_loss = (jnp.sum(qp0_ref[...] * (lse - s[0]))
               + jnp.sum(qp1_ref[...] * (lse - s[1]))
               + jnp.sum(qp2_ref[...] * (lse - s[2])))

    nhs = jnp.tanh(mm(x[NQ:], swn_ref[...]) + sbn_ref[...])
    chs = jnp.tanh(mm(ctx[NQ:], swc_ref[...]) + sbc_ref[...])
    ts = mm(chs, sb_ref[...])
    g = (jnp.sum(nhs * ts, axis=1, keepdims=True) + mm(nhs, swan_ref[...])
         + mm(chs, swac_ref[...]) + sba_ref[0:1, 0:1])
    gp = (jnp.maximum(g, 0.0) - g * sp_ref[...]
          + jnp.log1p(jnp.exp(-jnp.abs(g))))
    out_ref[...] = jnp.reshape(vr_loss + jnp.sum(gp), (1, 1))


def _tail_call(x, wv, z, question_prob, schema_prob, Wo, bo, qWn, qbn, qWc,
               qbc, qB, qWa, qba, sWn, sbn, sWc, sbc, sB, sWa, sba):
    ctx = pl.pallas_call(
        _ctx_body, out_shape=jax.ShapeDtypeStruct((N, H), jnp.float32))(
            wv, z, Wo, bo.reshape(1, H))
    out = pl.pallas_call(
        _loss_body, out_shape=jax.ShapeDtypeStruct((1, 1), jnp.float32))(
            x, ctx,
            question_prob[:, 0:1], question_prob[:, 1:2], question_prob[:, 2:3],
            schema_prob.reshape(NQ, 1),
            qWn, qbn.reshape(1, MH), qWc, qbc.reshape(1, MH), qB,
            qWa[:, :MH], qWa[:, MH:], qba.reshape(3, 1),
            sWn, sbn.reshape(1, MH), sWc, sbc.reshape(1, MH), sB[0],
            sWa[0:1, :MH], sWa[0:1, MH:], sba.reshape(1, 1))
    return out[0, 0]


def kernel(inputs, edge_index, question_prob, schema_prob, Wq, bq, Wk, Wv, Wo,
           bo, qWn, qbn, qWc, qbc, qB, qWa, qba, sWn, sbn, sWc, sbc, sB, sWa,
           sba):
    q, kv = _qkv_call(inputs, Wq, bq, Wk, Wv)
    npad = EP - E
    src_p = jnp.concatenate([edge_index[0], jnp.zeros((npad,), jnp.int32)])
    dst_p = jnp.concatenate([edge_index[1], jnp.full((npad,), N, jnp.int32)])
    out = _edge_call(src_p, dst_p, kv, q)
    wv = jnp.concatenate(
        [out[0, :NPW].reshape(NP, MH)[:N], out[1, :NPW].reshape(NP, MH)[:N]],
        axis=1)
    z = jnp.concatenate(
        [out[0, NPW:NPW + NZR].reshape(NP, HPC)[:N],
         out[1, NPW:NPW + NZR].reshape(NP, HPC)[:N]], axis=1)
    return _tail_call(inputs, wv, z, question_prob, schema_prob,
                      Wo, bo, qWn, qbn, qWc, qbc, qB, qWa, qba, sWn, sbn,
                      sWc, sbc, sB, sWa, sba)
